# Initial kernel scaffold; baseline (speedup 1.0000x reference)
import jax
import jax.numpy as jnp
from jax.experimental import pallas as pl


def kernel(node_features, adjacency, node_types, emb_table, proj_w, proj_b, lin_w, lin_b, att_w, att_b, pool_w1, pool_b1, pool_w2, pool_b2, cls_w1, cls_b1, cls_w2, cls_b2):
    raise NotImplementedError("write your pallas kernel here")


# speedup vs baseline: 2.0463x; 2.0463x over previous
import functools

import jax
import jax.numpy as jnp
from jax.experimental import pallas as pl

_B, _N, _D_FEAT, _HID, _LAYERS = 8, 256, 256, 256, 3
_N_TYPES, _N_CLASSES = 10, 8
_BN = _B * _N


def _dot_t(x, w):
    return jax.lax.dot_general(
        x, w, (((1,), (1,)), ((), ())), preferred_element_type=jnp.float32
    )


def _gnn_kernel(nf_ref, adj0_ref, nt_ref, emb_ref, projw_ref, projb_ref,
                linw_ref, linb_ref, attw_ref, attb_ref,
                pw1_ref, pb1_ref, pw2_ref,
                cw1_ref, cb1_ref, cw2_ref, cb2_ref,
                scores_ref, ge_ref):
    nf = nf_ref[...]
    nt = nt_ref[...]
    onehot = (nt == jax.lax.broadcasted_iota(jnp.int32, (_BN, _N_TYPES), 1)
              ).astype(jnp.float32)
    type_emb = jax.lax.dot_general(
        onehot, emb_ref[...], (((1,), (0,)), ((), ())),
        preferred_element_type=jnp.float32)
    feat_emb = _dot_t(nf, projw_ref[...]) + projb_ref[...]
    h = type_emb + feat_emb
    mask = (adj0_ref[...] > 0.0).astype(jnp.float32)

    for l in range(_LAYERS):
        t = _dot_t(h, linw_ref[l]) + linb_ref[l:l + 1, :]
        w1 = attw_ref[l:l + 1, :_HID]
        w2 = attw_ref[l:l + 1, _HID:]
        s1 = _dot_t(t, w1)
        s2 = jax.lax.dot_general(
            w2, t, (((1,), (1,)), ((), ())),
            preferred_element_type=jnp.float32)
        s2 = s2 + jax.lax.dot_general(
            attb_ref[l:l + 1, :], jnp.ones((1, _BN), jnp.float32),
            (((0,), (0,)), ((), ())), preferred_element_type=jnp.float32)
        s1mat = jax.lax.dot_general(
            s1, jnp.ones((1, _N), jnp.float32), (((1,), (0,)), ((), ())),
            preferred_element_type=jnp.float32)
        rows = []
        for g in range(_B):
            lo = g * _N
            t_g = t[lo:lo + _N, :]
            logits = s1mat[lo:lo + _N, :] + s2[:, lo:lo + _N]
            w = jax.nn.sigmoid(logits) * mask
            agg = jax.lax.dot_general(
                w, t_g, (((1,), (0,)), ((), ())),
                preferred_element_type=jnp.float32)
            rows.append(jax.nn.relu(t_g + agg))
        h = jnp.concatenate(rows, axis=0)

    ap = jnp.tanh(_dot_t(h, pw1_ref[...]) + pb1_ref[...])
    s = _dot_t(ap, pw2_ref[...])
    ges = []
    for g in range(_B):
        lo = g * _N
        s_g = s[lo:lo + _N, :]
        e = jnp.exp(s_g - jnp.max(s_g))
        a_g = e / jnp.sum(e)
        ges.append(jax.lax.dot_general(
            a_g, h[lo:lo + _N, :], (((0,), (0,)), ((), ())),
            preferred_element_type=jnp.float32))
    ge = jnp.concatenate(ges, axis=0)

    z = jax.nn.relu(_dot_t(ge, cw1_ref[...]) + cb1_ref[...])
    scores = _dot_t(z, cw2_ref[...]) + cb2_ref[...]

    scores_ref[...] = scores
    ge_ref[...] = ge


@jax.jit
def kernel(node_features, adjacency, node_types, emb_table, proj_w, proj_b,
           lin_w, lin_b, att_w, att_b, pool_w1, pool_b1, pool_w2, pool_b2,
           cls_w1, cls_b1, cls_w2, cls_b2):
    nf = node_features.reshape(_BN, _D_FEAT)
    adj0 = adjacency[0]
    nt = node_types.reshape(_BN, 1).astype(jnp.int32)
    scores, ge = pl.pallas_call(
        _gnn_kernel,
        out_shape=[
            jax.ShapeDtypeStruct((_B, _N_CLASSES), jnp.float32),
            jax.ShapeDtypeStruct((_B, _HID), jnp.float32),
        ],
    )(nf, adj0, nt, emb_table, proj_w, proj_b.reshape(1, _HID),
      lin_w, lin_b, att_w.reshape(_LAYERS, 2 * _HID), att_b,
      pool_w1, pool_b1.reshape(1, _HID // 2), pool_w2,
      cls_w1, cls_b1.reshape(1, _HID // 2),
      cls_w2, cls_b2.reshape(1, _N_CLASSES))
    return (scores, ge)
